# baseline (speedup 1.0000x reference)
import jax
import jax.numpy as jnp
from jax.experimental import pallas as pl
from jax.experimental.pallas import tpu as pltpu

B, S, D, H, G, Dh, E, HD, TK = 32, 8, 1024, 16, 4, 64, 64, 512, 2
T = B * S
EPS = 1e-06
NEG = -1e30
NBUF = 4


def _rms(x, w):
    rms = jnp.sqrt(jnp.mean(x * x, axis=-1, keepdims=True))
    return x / (rms + EPS) * w


def _fused_kernel(pos_ref, x_ref, n1_ref, n2_ref, wq_ref, wk_ref, wv_ref,
                  wo_ref, qn_ref, kn_ref, gw_ref, wg_hbm, wu_hbm, wd_hbm,
                  out_ref, kout_ref, vout_ref,
                  ctx_ref, h2_ref, wdense_ref, wgbuf, wubuf, wdbuf, sems):

    def slot_copies(slot, e):
        return (
            pltpu.make_async_copy(wg_hbm.at[e], wgbuf.at[slot], sems.at[slot]),
            pltpu.make_async_copy(wu_hbm.at[e], wubuf.at[slot], sems.at[slot]),
            pltpu.make_async_copy(wd_hbm.at[e], wdbuf.at[slot], sems.at[slot]),
        )

    for i in range(NBUF):
        for cp in slot_copies(i, i):
            cp.start()

    x = x_ref[...]
    h = _rms(x, n1_ref[...])

    q_all = jnp.dot(h, wq_ref[...], preferred_element_type=jnp.float32)
    k_all = jnp.dot(h, wk_ref[...], preferred_element_type=jnp.float32)
    v_all = jnp.dot(h, wv_ref[...], preferred_element_type=jnp.float32)

    c_iota = jax.lax.broadcasted_iota(jnp.int32, (T, Dh), 1)
    r_iota = jax.lax.broadcasted_iota(jnp.int32, (T, Dh), 0)
    j = (c_iota % (Dh // 2)).astype(jnp.float32)
    inv_freq = jnp.exp(j * (-jnp.log(10000.0) * 2.0 / Dh))
    rmod = r_iota % S
    p = jnp.zeros((T, Dh), jnp.float32)
    for s in range(S):
        p = jnp.where(rmod == s, pos_ref[0, s].astype(jnp.float32), p)
    emb = p * inv_freq
    cos_t = jnp.cos(emb)
    sin_t = jnp.sin(emb)

    def rope(z):
        z1 = z[:, : Dh // 2]
        z2 = z[:, Dh // 2:]
        rot = jnp.concatenate([-z2, z1], axis=1)
        return z * cos_t + rot * sin_t

    rr = jax.lax.broadcasted_iota(jnp.int32, (T, T), 0)
    cc = jax.lax.broadcasted_iota(jnp.int32, (T, T), 1)
    valid = ((rr // S) == (cc // S)) & (cc <= rr)

    kn = kn_ref[...]
    qn = qn_ref[...]

    kv = []
    for g in range(G):
        kh = rope(_rms(k_all[:, g * Dh:(g + 1) * Dh], kn))
        vh = v_all[:, g * Dh:(g + 1) * Dh]
        kout_ref[:, g * Dh:(g + 1) * Dh] = kh
        vout_ref[:, g * Dh:(g + 1) * Dh] = vh
        kv.append((kh, vh))

    for hh in range(H):
        qh = rope(_rms(q_all[:, hh * Dh:(hh + 1) * Dh], qn))
        kh, vh = kv[hh // (H // G)]
        scores = jax.lax.dot_general(
            qh, kh, (((1,), (1,)), ((), ())),
            preferred_element_type=jnp.float32) * (1.0 / (Dh ** 0.5))
        scores = jnp.where(valid, scores, NEG)
        m = jnp.max(scores, axis=1, keepdims=True)
        ex = jnp.exp(scores - m)
        attn = ex / jnp.sum(ex, axis=1, keepdims=True)
        ctx_ref[:, hh * Dh:(hh + 1) * Dh] = jnp.dot(
            attn, vh, preferred_element_type=jnp.float32)

    x1 = jnp.dot(ctx_ref[...], wo_ref[...],
                 preferred_element_type=jnp.float32) + x
    out_ref[...] = x1
    h2 = _rms(x1, n2_ref[...])
    h2_ref[...] = h2

    logits = jnp.dot(h2, gw_ref[...], preferred_element_type=jnp.float32)
    lanes = jax.lax.broadcasted_iota(jnp.int32, (T, E), 1)
    m1 = jnp.max(logits, axis=1, keepdims=True)
    i1 = jnp.min(jnp.where(logits == m1, lanes, E), axis=1, keepdims=True)
    masked = jnp.where(lanes == i1, NEG, logits)
    m2 = jnp.max(masked, axis=1, keepdims=True)
    i2 = jnp.min(jnp.where(masked == m2, lanes, E), axis=1, keepdims=True)
    d = jnp.exp(m2 - m1)
    w1 = 1.0 / (1.0 + d)
    w2 = d / (1.0 + d)
    wdense_ref[...] = (jnp.where(lanes == i1, w1, 0.0)
                       + jnp.where(lanes == i2, w2, 0.0))

    def body(i, carry):
        e0 = i * NBUF
        for k in range(NBUF):
            e = e0 + k
            for cp in slot_copies(k, e):
                cp.wait()

            g = jnp.dot(h2, wgbuf[k], preferred_element_type=jnp.float32)
            u = jnp.dot(h2, wubuf[k], preferred_element_type=jnp.float32)
            g = g * (1.0 / (1.0 + jnp.exp(-g)))
            wcol = jnp.sum(jnp.where(lanes == e, wdense_ref[...], 0.0),
                           axis=1, keepdims=True)
            eo = jnp.dot(wcol * (g * u), wdbuf[k],
                         preferred_element_type=jnp.float32)
            out_ref[...] += eo

            @pl.when(e + NBUF < E)
            def _next():
                for cp in slot_copies(k, e + NBUF):
                    cp.start()

        return carry

    jax.lax.fori_loop(0, E // NBUF, body, 0)


@jax.jit
def kernel(x, position_ids, norm1_w, norm2_w, Wq, Wk, Wv, Wo, q_norm_w,
           k_norm_w, gate_W, Wg, Wu, Wd):
    xf = x.reshape(T, D)

    out, k_flat, v_flat = pl.pallas_call(
        _fused_kernel,
        grid=(),
        in_specs=[
            pl.BlockSpec(memory_space=pltpu.SMEM),
            pl.BlockSpec(memory_space=pltpu.VMEM),
            pl.BlockSpec(memory_space=pltpu.VMEM),
            pl.BlockSpec(memory_space=pltpu.VMEM),
            pl.BlockSpec(memory_space=pltpu.VMEM),
            pl.BlockSpec(memory_space=pltpu.VMEM),
            pl.BlockSpec(memory_space=pltpu.VMEM),
            pl.BlockSpec(memory_space=pltpu.VMEM),
            pl.BlockSpec(memory_space=pltpu.VMEM),
            pl.BlockSpec(memory_space=pltpu.VMEM),
            pl.BlockSpec(memory_space=pltpu.VMEM),
            pl.BlockSpec(memory_space=pl.ANY),
            pl.BlockSpec(memory_space=pl.ANY),
            pl.BlockSpec(memory_space=pl.ANY),
        ],
        out_specs=[
            pl.BlockSpec(memory_space=pltpu.VMEM),
            pl.BlockSpec(memory_space=pltpu.VMEM),
            pl.BlockSpec(memory_space=pltpu.VMEM),
        ],
        out_shape=[
            jax.ShapeDtypeStruct((T, D), jnp.float32),
            jax.ShapeDtypeStruct((T, G * Dh), jnp.float32),
            jax.ShapeDtypeStruct((T, G * Dh), jnp.float32),
        ],
        scratch_shapes=[
            pltpu.VMEM((T, H * Dh), jnp.float32),
            pltpu.VMEM((T, D), jnp.float32),
            pltpu.VMEM((T, E), jnp.float32),
            pltpu.VMEM((NBUF, D, HD), jnp.float32),
            pltpu.VMEM((NBUF, D, HD), jnp.float32),
            pltpu.VMEM((NBUF, HD, D), jnp.float32),
            pltpu.SemaphoreType.DMA((NBUF,)),
        ],
    )(position_ids.reshape(1, S), xf, norm1_w.reshape(1, D),
      norm2_w.reshape(1, D), Wq, Wk, Wv, Wo, q_norm_w.reshape(1, Dh),
      k_norm_w.reshape(1, Dh), gate_W, Wg, Wu, Wd)

    new_k = k_flat.reshape(B, S, G, Dh).transpose(0, 2, 1, 3)
    new_v = v_flat.reshape(B, S, G, Dh).transpose(0, 2, 1, 3)
    return out.reshape(B, S, D), new_k, new_v


# speedup vs baseline: 1.0379x; 1.0379x over previous
import jax
import jax.numpy as jnp
from jax.experimental import pallas as pl
from jax.experimental.pallas import tpu as pltpu

B, S, D, H, G, Dh, E, HD, TK = 32, 8, 1024, 16, 4, 64, 64, 512, 2
T = B * S
EPS = 1e-06
NEG = -1e30
NBUF = 4


def _rms(x, w):
    rms = jnp.sqrt(jnp.mean(x * x, axis=-1, keepdims=True))
    return x / (rms + EPS) * w


def _fused_kernel(pos_ref, x_ref, n1_ref, n2_ref, wq_ref, wk_ref, wv_ref,
                  wo_ref, qn_ref, kn_ref, gw_ref, wg_hbm, wu_hbm, wd_hbm,
                  out_ref, kout_ref, vout_ref,
                  ctx_ref, h2_ref, wdense_ref, wgbuf, wubuf, wdbuf, sems):

    def slot_copies(slot, e):
        return (
            pltpu.make_async_copy(wg_hbm.at[e], wgbuf.at[slot], sems.at[slot]),
            pltpu.make_async_copy(wu_hbm.at[e], wubuf.at[slot], sems.at[slot]),
            pltpu.make_async_copy(wd_hbm.at[e], wdbuf.at[slot], sems.at[slot]),
        )

    for i in range(NBUF):
        for cp in slot_copies(i, i):
            cp.start()

    x = x_ref[...]
    h = _rms(x, n1_ref[...])

    q_all = jnp.dot(h, wq_ref[...], preferred_element_type=jnp.float32)
    k_all = jnp.dot(h, wk_ref[...], preferred_element_type=jnp.float32)
    v_all = jnp.dot(h, wv_ref[...], preferred_element_type=jnp.float32)

    c_iota = jax.lax.broadcasted_iota(jnp.int32, (T, Dh), 1)
    r_iota = jax.lax.broadcasted_iota(jnp.int32, (T, Dh), 0)
    j = (c_iota % (Dh // 2)).astype(jnp.float32)
    inv_freq = jnp.exp(j * (-jnp.log(10000.0) * 2.0 / Dh))
    rmod = r_iota % S
    p = jnp.zeros((T, Dh), jnp.float32)
    for s in range(S):
        p = jnp.where(rmod == s, pos_ref[0, s].astype(jnp.float32), p)
    emb = p * inv_freq
    cos_t = jnp.cos(emb)
    sin_t = jnp.sin(emb)

    def headnorm(z, wrow):
        dd = z.shape[1]
        nh = dd // Dh
        di = jax.lax.broadcasted_iota(jnp.int32, (dd, 128), 0)
        hi = jax.lax.broadcasted_iota(jnp.int32, (dd, 128), 1)
        bd = (di // Dh == hi).astype(jnp.float32)
        dit = jax.lax.broadcasted_iota(jnp.int32, (128, dd), 1)
        hit = jax.lax.broadcasted_iota(jnp.int32, (128, dd), 0)
        bdt = (dit // Dh == hit).astype(jnp.float32)
        ssq = jnp.dot(z * z, bd, preferred_element_type=jnp.float32)
        rms = jnp.sqrt(ssq * (1.0 / Dh))
        rmap = jnp.dot(rms, bdt, preferred_element_type=jnp.float32)
        wt = jnp.concatenate([wrow] * nh, axis=1)
        return z / (rmap + EPS) * wt

    def rope_all(z):
        dd = z.shape[1]
        nh = dd // Dh
        cos_a = jnp.concatenate([cos_t] * nh, axis=1)
        sin_a = jnp.concatenate([sin_t] * nh, axis=1)
        a = jnp.concatenate([z[:, 32:], z[:, :32]], axis=1)
        b = jnp.concatenate([z[:, -32:], z[:, :-32]], axis=1)
        lm = jax.lax.broadcasted_iota(jnp.int32, (T, dd), 1) % Dh
        rot = jnp.where(lm < 32, -a, b)
        return z * cos_a + rot * sin_a

    rr = jax.lax.broadcasted_iota(jnp.int32, (T, T), 0)
    cc = jax.lax.broadcasted_iota(jnp.int32, (T, T), 1)
    valid = ((rr // S) == (cc // S)) & (cc <= rr)

    q_proc = rope_all(headnorm(q_all, qn_ref[...]))
    k_proc = rope_all(headnorm(k_all, kn_ref[...]))
    kout_ref[...] = k_proc
    vout_ref[...] = v_all

    for hh in range(H):
        qh = q_proc[:, hh * Dh:(hh + 1) * Dh]
        g = hh // (H // G)
        kh = k_proc[:, g * Dh:(g + 1) * Dh]
        vh = v_all[:, g * Dh:(g + 1) * Dh]
        scores = jax.lax.dot_general(
            qh, kh, (((1,), (1,)), ((), ())),
            preferred_element_type=jnp.float32) * (1.0 / (Dh ** 0.5))
        scores = jnp.where(valid, scores, NEG)
        m = jnp.max(scores, axis=1, keepdims=True)
        ex = jnp.exp(scores - m)
        attn = ex / jnp.sum(ex, axis=1, keepdims=True)
        ctx_ref[:, hh * Dh:(hh + 1) * Dh] = jnp.dot(
            attn, vh, preferred_element_type=jnp.float32)

    x1 = jnp.dot(ctx_ref[...], wo_ref[...],
                 preferred_element_type=jnp.float32) + x
    out_ref[...] = x1
    h2 = _rms(x1, n2_ref[...])
    h2_ref[...] = h2

    logits = jnp.dot(h2, gw_ref[...], preferred_element_type=jnp.float32)
    lanes = jax.lax.broadcasted_iota(jnp.int32, (T, E), 1)
    m1 = jnp.max(logits, axis=1, keepdims=True)
    i1 = jnp.min(jnp.where(logits == m1, lanes, E), axis=1, keepdims=True)
    masked = jnp.where(lanes == i1, NEG, logits)
    m2 = jnp.max(masked, axis=1, keepdims=True)
    i2 = jnp.min(jnp.where(masked == m2, lanes, E), axis=1, keepdims=True)
    d = jnp.exp(m2 - m1)
    w1 = 1.0 / (1.0 + d)
    w2 = d / (1.0 + d)
    wdense_ref[...] = (jnp.where(lanes == i1, w1, 0.0)
                       + jnp.where(lanes == i2, w2, 0.0))

    def body(i, carry):
        e0 = i * NBUF
        for k in range(NBUF):
            e = e0 + k
            for cp in slot_copies(k, e):
                cp.wait()

            g = jnp.dot(h2, wgbuf[k], preferred_element_type=jnp.float32)
            u = jnp.dot(h2, wubuf[k], preferred_element_type=jnp.float32)
            g = g * (1.0 / (1.0 + jnp.exp(-g)))
            wcol = jnp.sum(jnp.where(lanes == e, wdense_ref[...], 0.0),
                           axis=1, keepdims=True)
            eo = jnp.dot(wcol * (g * u), wdbuf[k],
                         preferred_element_type=jnp.float32)
            out_ref[...] += eo

            @pl.when(e + NBUF < E)
            def _next():
                for cp in slot_copies(k, e + NBUF):
                    cp.start()

        return carry

    jax.lax.fori_loop(0, E // NBUF, body, 0)


@jax.jit
def kernel(x, position_ids, norm1_w, norm2_w, Wq, Wk, Wv, Wo, q_norm_w,
           k_norm_w, gate_W, Wg, Wu, Wd):
    xf = x.reshape(T, D)

    out, k_flat, v_flat = pl.pallas_call(
        _fused_kernel,
        grid=(),
        in_specs=[
            pl.BlockSpec(memory_space=pltpu.SMEM),
            pl.BlockSpec(memory_space=pltpu.VMEM),
            pl.BlockSpec(memory_space=pltpu.VMEM),
            pl.BlockSpec(memory_space=pltpu.VMEM),
            pl.BlockSpec(memory_space=pltpu.VMEM),
            pl.BlockSpec(memory_space=pltpu.VMEM),
            pl.BlockSpec(memory_space=pltpu.VMEM),
            pl.BlockSpec(memory_space=pltpu.VMEM),
            pl.BlockSpec(memory_space=pltpu.VMEM),
            pl.BlockSpec(memory_space=pltpu.VMEM),
            pl.BlockSpec(memory_space=pltpu.VMEM),
            pl.BlockSpec(memory_space=pl.ANY),
            pl.BlockSpec(memory_space=pl.ANY),
            pl.BlockSpec(memory_space=pl.ANY),
        ],
        out_specs=[
            pl.BlockSpec(memory_space=pltpu.VMEM),
            pl.BlockSpec(memory_space=pltpu.VMEM),
            pl.BlockSpec(memory_space=pltpu.VMEM),
        ],
        out_shape=[
            jax.ShapeDtypeStruct((T, D), jnp.float32),
            jax.ShapeDtypeStruct((T, G * Dh), jnp.float32),
            jax.ShapeDtypeStruct((T, G * Dh), jnp.float32),
        ],
        scratch_shapes=[
            pltpu.VMEM((T, H * Dh), jnp.float32),
            pltpu.VMEM((T, D), jnp.float32),
            pltpu.VMEM((T, E), jnp.float32),
            pltpu.VMEM((NBUF, D, HD), jnp.float32),
            pltpu.VMEM((NBUF, D, HD), jnp.float32),
            pltpu.VMEM((NBUF, HD, D), jnp.float32),
            pltpu.SemaphoreType.DMA((NBUF,)),
        ],
    )(position_ids.reshape(1, S), xf, norm1_w.reshape(1, D),
      norm2_w.reshape(1, D), Wq, Wk, Wv, Wo, q_norm_w.reshape(1, Dh),
      k_norm_w.reshape(1, Dh), gate_W, Wg, Wu, Wd)

    new_k = k_flat.reshape(B, S, G, Dh).transpose(0, 2, 1, 3)
    new_v = v_flat.reshape(B, S, G, Dh).transpose(0, 2, 1, 3)
    return out.reshape(B, S, D), new_k, new_v
